# baseline (speedup 1.0000x reference)
import functools

import jax
import jax.numpy as jnp
from jax import lax
from jax.experimental import pallas as pl
from jax.experimental.pallas import tpu as pltpu
from jax.experimental.pallas import tpu_sc as plsc

B = 16384
N = 11
D = 256
R = B * N
NC = 2
NS = 16
NW = NC * NS
RPW = R // NW
CH = 176
NCHUNK = RPW // CH
TF = 2 * N * D

_DNUMS = lax.GatherDimensionNumbers(
    offset_dims=(), collapsed_slice_dims=(0,), start_index_map=(0,))


def _sc_body(c_hbm, t_hbm, out_hbm, c_v, t_v, buf0, buf1, s0, s1):
    cid = lax.axis_index("c")
    sid = lax.axis_index("s")
    wid = sid * NC + cid
    base = wid * RPW

    pltpu.sync_copy(c_hbm.at[pl.ds(base, RPW)], c_v.at[pl.ds(0, RPW)])
    pltpu.sync_copy(t_hbm, t_v)

    iota = lax.iota(jnp.int32, 16)

    def compute(j, buf):
        g0 = j * CH

        def bat_body(bi, carry):
            g = g0 + bi * N
            cv = c_v[pl.ds(g, 16)]
            for n in range(N):
                cs = cv[n]
                off = cs * (N * D) + n * D
                row_off = (bi * N + n) * D
                for k in range(D // 16):
                    buf[pl.ds(row_off + k * 16, 16)] = t_v[pl.ds(off + k * 16, 16)]
            return carry

        lax.fori_loop(0, CH // N, bat_body, 0)

    def scat(j, buf, sem):
        pltpu.async_copy(buf, out_hbm.at[pl.ds((base + j * CH) * D, CH * D)], sem)

    def scat_wait(buf, sem):
        pltpu.make_async_copy(buf, out_hbm.at[pl.ds(base * D, CH * D)], sem).wait()

    compute(0, buf0)
    scat(0, buf0, s0)
    compute(1, buf1)
    scat(1, buf1, s1)

    def pair_body(p, carry):
        j0 = p * 2
        scat_wait(buf0, s0)
        compute(j0, buf0)
        scat(j0, buf0, s0)
        scat_wait(buf1, s1)
        compute(j0 + 1, buf1)
        scat(j0 + 1, buf1, s1)
        return carry

    lax.fori_loop(1, NCHUNK // 2, pair_body, 0)
    scat_wait(buf0, s0)
    scat_wait(buf1, s1)


_sc_encode = functools.partial(
    pl.kernel,
    mesh=plsc.VectorSubcoreMesh(core_axis_name="c", subcore_axis_name="s"),
    out_type=jax.ShapeDtypeStruct((R * D,), jnp.float32),
    compiler_params=pltpu.CompilerParams(needs_layout_passes=False),
    scratch_types=[
        pltpu.VMEM((RPW + 16,), jnp.int32),
        pltpu.VMEM((TF,), jnp.float32),
        pltpu.VMEM((CH * D,), jnp.float32),
        pltpu.VMEM((CH * D,), jnp.float32),
        pltpu.SemaphoreType.DMA,
        pltpu.SemaphoreType.DMA,
    ],
)(_sc_body)


def kernel(c, attr_embed, null_embed):
    table = jnp.concatenate([null_embed, attr_embed], axis=0).reshape(TF)
    out = _sc_encode(c.reshape(R), table)
    return out.reshape(B, N, D)


# speedup vs baseline: 1.0951x; 1.0951x over previous
import functools

import jax
import jax.numpy as jnp
from jax import lax
from jax.experimental import pallas as pl
from jax.experimental.pallas import tpu as pltpu
from jax.experimental.pallas import tpu_sc as plsc

B = 16384
N = 11
D = 256
R = B * N
NC = 2
NS = 16
NW = NC * NS
RPW = R // NW
CH = 128
NCHUNK = RPW // CH
NQUAD = NCHUNK // 4
TF = 2 * N * D

_DNUMS = lax.GatherDimensionNumbers(
    offset_dims=(), collapsed_slice_dims=(0,), start_index_map=(0,))


def _sc_body(c_hbm, t_hbm, tf_hbm, out_hbm, c_v, idx_v, t_v, bufa, bufb, bufc,
             sa, sb, sc, gb):
    cid = lax.axis_index("c")
    sid = lax.axis_index("s")
    wid = sid * NC + cid
    base = wid * RPW

    pltpu.sync_copy(c_hbm.at[pl.ds(base, RPW)], c_v)
    pltpu.sync_copy(tf_hbm, t_v)

    iota = lax.iota(jnp.int32, 16)

    def idx_body(v, carry):
        offs = v * 16
        cvec = c_v[pl.ds(offs, 16)]
        nvec = lax.rem(offs + iota, N)
        idx_v[v // 8, pl.ds((v % 8) * 16, 16)] = nvec + cvec * N
        return carry

    lax.fori_loop(0, RPW // 16, idx_body, 0)

    def compute(j, buf):
        g0 = j * CH

        def grp_body(q, carry):
            i0 = q * 16
            civ = c_v[pl.ds(g0 + i0, 16)]
            nv = lax.rem(g0 + i0 + iota, N)
            rowbase = (nv + civ * N) * D
            for r in range(16):
                fb = lax.gather(
                    rowbase, jnp.full((16, 1), r, jnp.int32), _DNUMS, (1,),
                    mode=lax.GatherScatterMode.PROMISE_IN_BOUNDS)
                row = i0 + r
                for k in range(D // 16):
                    vals = plsc.load_gather(t_v, [fb + (iota + k * 16)])
                    buf[row, pl.ds(k * 16, 16)] = vals
            return carry

        lax.fori_loop(0, CH // 16, grp_body, 0)

    def gath(j, buf, sem):
        pltpu.async_copy(t_hbm.at[idx_v.at[j]], buf, sem)

    def gath_wait(buf, sem):
        pltpu.make_async_copy(t_hbm.at[idx_v.at[1]], buf, sem).wait()

    def scat(j, buf, sem):
        pltpu.async_copy(buf, out_hbm.at[pl.ds(base + j * CH, CH)], sem)

    def scat_wait(buf, sem):
        pltpu.make_async_copy(buf, out_hbm.at[pl.ds(base, CH)], sem).wait()

    def quad(q, primed):
        j = q * 4
        if primed:
            scat_wait(bufb, sb)
        gath(j + 1, bufb, gb)
        if primed:
            scat_wait(bufa, sa)
        compute(j, bufa)
        scat(j, bufa, sa)
        gath_wait(bufb, gb)
        scat(j + 1, bufb, sb)
        if primed:
            scat_wait(bufc, sc)
        compute(j + 2, bufc)
        scat(j + 2, bufc, sc)
        scat_wait(bufa, sa)
        compute(j + 3, bufa)
        scat(j + 3, bufa, sa)

    quad(0, False)

    def quad_body(q, carry):
        quad(q, True)
        return carry

    lax.fori_loop(1, NQUAD, quad_body, 0)
    scat_wait(bufa, sa)
    scat_wait(bufb, sb)
    scat_wait(bufc, sc)


_sc_encode = functools.partial(
    pl.kernel,
    mesh=plsc.VectorSubcoreMesh(core_axis_name="c", subcore_axis_name="s"),
    out_type=jax.ShapeDtypeStruct((R, D), jnp.float32),
    compiler_params=pltpu.CompilerParams(needs_layout_passes=False),
    scratch_types=[
        pltpu.VMEM((RPW,), jnp.int32),
        pltpu.VMEM((NCHUNK, CH), jnp.int32),
        pltpu.VMEM((TF,), jnp.float32),
        pltpu.VMEM((CH, D), jnp.float32),
        pltpu.VMEM((CH, D), jnp.float32),
        pltpu.VMEM((CH, D), jnp.float32),
        pltpu.SemaphoreType.DMA,
        pltpu.SemaphoreType.DMA,
        pltpu.SemaphoreType.DMA,
        pltpu.SemaphoreType.DMA,
    ],
)(_sc_body)


def kernel(c, attr_embed, null_embed):
    table = jnp.concatenate([null_embed, attr_embed], axis=0)
    out = _sc_encode(c.reshape(R), table, table.reshape(TF))
    return out.reshape(B, N, D)


# speedup vs baseline: 1.3994x; 1.2779x over previous
import functools

import jax
import jax.numpy as jnp
from jax import lax
from jax.experimental import pallas as pl
from jax.experimental.pallas import tpu as pltpu
from jax.experimental.pallas import tpu_sc as plsc

B = 16384
N = 11
D = 256
R = B * N
NC = 2
NS = 16
NW = NC * NS
RPW = R // NW
CH = 88
NCHUNK = RPW // CH
TF = 2 * N * D

_DNUMS = lax.GatherDimensionNumbers(
    offset_dims=(), collapsed_slice_dims=(0,), start_index_map=(0,))


def _sc_body(c_hbm, t_hbm, out_hbm, c_v, t_v, buf0, buf1, s0, s1):
    cid = lax.axis_index("c")
    sid = lax.axis_index("s")
    wid = sid * NC + cid
    base = wid * RPW
    bbase = wid * (B // NW)

    pltpu.sync_copy(c_hbm.at[pl.ds(base, RPW)], c_v.at[pl.ds(0, RPW)])
    pltpu.sync_copy(t_hbm, t_v)

    iota = lax.iota(jnp.int32, 16)

    def compute(j, buf):
        g0 = j * CH

        def bat_body(bi, carry):
            g = g0 + bi * N
            cv = c_v[pl.ds(g, 16)]
            for n in range(N):
                cs = cv[n]
                off = cs * (N * D) + n * D
                for k in range(D // 16):
                    buf[bi, n, pl.ds(k * 16, 16)] = t_v[pl.ds(off + k * 16, 16)]
            return carry

        lax.fori_loop(0, CH // N, bat_body, 0)

    BPC = CH // N

    def scat(j, buf, sem):
        pltpu.async_copy(buf, out_hbm.at[pl.ds(bbase + j * BPC, BPC)], sem)

    def scat_wait(buf, sem):
        pltpu.make_async_copy(buf, out_hbm.at[pl.ds(bbase, BPC)], sem).wait()

    compute(0, buf0)
    scat(0, buf0, s0)
    compute(1, buf1)
    scat(1, buf1, s1)

    def pair_body(p, carry):
        j0 = p * 2
        scat_wait(buf0, s0)
        compute(j0, buf0)
        scat(j0, buf0, s0)
        scat_wait(buf1, s1)
        compute(j0 + 1, buf1)
        scat(j0 + 1, buf1, s1)
        return carry

    lax.fori_loop(1, NCHUNK // 2, pair_body, 0)
    scat_wait(buf0, s0)
    scat_wait(buf1, s1)


_sc_encode = functools.partial(
    pl.kernel,
    mesh=plsc.VectorSubcoreMesh(core_axis_name="c", subcore_axis_name="s"),
    out_type=jax.ShapeDtypeStruct((B, N, D), jnp.float32),
    compiler_params=pltpu.CompilerParams(needs_layout_passes=False),
    scratch_types=[
        pltpu.VMEM((RPW + 16,), jnp.int32),
        pltpu.VMEM((TF,), jnp.float32),
        pltpu.VMEM((CH // N, N, D), jnp.float32),
        pltpu.VMEM((CH // N, N, D), jnp.float32),
        pltpu.SemaphoreType.DMA,
        pltpu.SemaphoreType.DMA,
    ],
)(_sc_body)


def kernel(c, attr_embed, null_embed):
    table = jnp.concatenate([null_embed, attr_embed], axis=0).reshape(TF)
    return _sc_encode(c.reshape(R), table)


# speedup vs baseline: 2.0188x; 1.4426x over previous
import functools

import jax
import jax.numpy as jnp
from jax import lax
from jax.experimental import pallas as pl
from jax.experimental.pallas import tpu as pltpu
from jax.experimental.pallas import tpu_sc as plsc

B = 16384
N = 11
D = 256
R = B * N
NC = 2
NS = 16
NW = NC * NS
RPW = R // NW
CH = 88
NCHUNK = RPW // CH
TF = 2 * N * D

_DNUMS = lax.GatherDimensionNumbers(
    offset_dims=(), collapsed_slice_dims=(0,), start_index_map=(0,))


def _sc_body(c_hbm, t_hbm, out_hbm, c_v, t_v, buf0, buf1, s0, s1):
    cid = lax.axis_index("c")
    sid = lax.axis_index("s")
    wid = sid * NC + cid
    base = wid * RPW
    bbase = wid * (B // NW)

    pltpu.sync_copy(c_hbm.at[pl.ds(base, RPW)], c_v.at[pl.ds(0, RPW)])
    pltpu.sync_copy(t_hbm, t_v)

    iota = lax.iota(jnp.int32, 16)

    def compute(j, buf):
        g0 = j * CH

        for n in range(N):
            for kb in range(2):
                nulls = [t_v[pl.ds(n * D + kb * 128 + k * 16, 16)]
                         for k in range(8)]
                attrs = [t_v[pl.ds((N + n) * D + kb * 128 + k * 16, 16)]
                         for k in range(8)]

                def bi_body(bi, carry, n=n, kb=kb, nulls=nulls, attrs=attrs):
                    cv = c_v[pl.ds(g0 + bi * N, 16)]
                    m = lax.broadcast_in_dim(cv[n], (16,), ()) != 0
                    for k in range(8):
                        buf[bi, n, pl.ds(kb * 128 + k * 16, 16)] = (
                            jnp.where(m, attrs[k], nulls[k]))
                    return carry

                lax.fori_loop(0, CH // N, bi_body, 0)

    BPC = CH // N

    def scat(j, buf, sem):
        pltpu.async_copy(buf, out_hbm.at[pl.ds(bbase + j * BPC, BPC)], sem)

    def scat_wait(buf, sem):
        pltpu.make_async_copy(buf, out_hbm.at[pl.ds(bbase, BPC)], sem).wait()

    compute(0, buf0)
    scat(0, buf0, s0)
    compute(1, buf1)
    scat(1, buf1, s1)

    def pair_body(p, carry):
        j0 = p * 2
        scat_wait(buf0, s0)
        compute(j0, buf0)
        scat(j0, buf0, s0)
        scat_wait(buf1, s1)
        compute(j0 + 1, buf1)
        scat(j0 + 1, buf1, s1)
        return carry

    lax.fori_loop(1, NCHUNK // 2, pair_body, 0)
    scat_wait(buf0, s0)
    scat_wait(buf1, s1)


_sc_encode = functools.partial(
    pl.kernel,
    mesh=plsc.VectorSubcoreMesh(core_axis_name="c", subcore_axis_name="s"),
    out_type=jax.ShapeDtypeStruct((B, N, D), jnp.float32),
    compiler_params=pltpu.CompilerParams(needs_layout_passes=False),
    scratch_types=[
        pltpu.VMEM((RPW + 16,), jnp.int32),
        pltpu.VMEM((TF,), jnp.float32),
        pltpu.VMEM((CH // N, N, D), jnp.float32),
        pltpu.VMEM((CH // N, N, D), jnp.float32),
        pltpu.SemaphoreType.DMA,
        pltpu.SemaphoreType.DMA,
    ],
)(_sc_body)


def kernel(c, attr_embed, null_embed):
    table = jnp.concatenate([null_embed, attr_embed], axis=0).reshape(TF)
    return _sc_encode(c.reshape(R), table)
